# baseline (speedup 1.0000x reference)
import functools

import jax
import jax.numpy as jnp
from jax import lax
from jax.experimental import pallas as pl
from jax.experimental.pallas import tpu as pltpu
from jax.experimental.pallas import tpu_sc as plsc

_DELTA_VAR = 0.5
_DELTA_DIST = 1.5
_ALPHA = 1.0
_BETA = 1.0
_GAMMA = 0.001
_K = 32
_C = 32
_CHUNK = 1024
_GROUPS = _CHUNK // 16
_ACC = 1088
_ACC4 = _ACC * 4


def _vsqrt(x):
    i = plsc.bitcast(x, jnp.int32)
    i = jnp.int32(0x5F3759DF) - lax.shift_right_logical(i, 1)
    y = plsc.bitcast(i, jnp.float32)
    for _ in range(2):
        y = y * (1.5 - 0.5 * x * y * y)
    return x * y


def _sc_loss_body(nchunks, x_hbm, ids_hbm, out_hbm,
                  xbuf, idbuf, accbuf, accbuf2, accbuf3, accbuf4, stage,
                  meanflat, meanflat4, mtflat, redbuf, statall,
                  obuf, semx, semi, shared_all, shared_m):
    cid = lax.axis_index("c")
    sid = lax.axis_index("s")
    b = cid
    tile_px = nchunks * _CHUNK
    base = sid * tile_px

    zeros16 = jnp.zeros((16,), jnp.float32)
    ones16 = jnp.ones((16,), jnp.float32)
    iota16 = lax.iota(jnp.int32, 16)

    accbufs = (accbuf, accbuf2, accbuf3, accbuf4)

    def zero_body(j, _):
        sl = pl.ds(j * 16, 16)
        accbuf[sl] = zeros16
        accbuf2[sl] = zeros16
        accbuf3[sl] = zeros16
        accbuf4[sl] = zeros16
        return 0

    lax.fori_loop(0, _ACC4 // 16, zero_body, 0)

    def start_fetch(ch, slot):
        off = base + ch * _CHUNK
        pltpu.async_copy(x_hbm.at[b, :, pl.ds(off, _CHUNK)], xbuf.at[slot],
                         semx.at[slot])
        pltpu.async_copy(ids_hbm.at[b, pl.ds(off, _CHUNK)], idbuf.at[slot],
                         semi.at[slot])

    def wait_fetch(ch, slot):
        off = base + ch * _CHUNK
        pltpu.make_async_copy(x_hbm.at[b, :, pl.ds(off, _CHUNK)],
                              xbuf.at[slot], semx.at[slot]).wait()
        pltpu.make_async_copy(ids_hbm.at[b, pl.ds(off, _CHUNK)],
                              idbuf.at[slot], semi.at[slot]).wait()

    def run_pass(process_group):
        start_fetch(0, 0)

        def chunk_body(ch, _):
            slot = lax.bitwise_and(ch, 1)
            nslot = 1 - slot

            @pl.when(ch + 1 < nchunks)
            def _():
                start_fetch(ch + 1, nslot)

            wait_fetch(ch, slot)

            @plsc.parallel_loop(0, _GROUPS, unroll=16)
            def _(g):
                process_group(slot, g * 16)

            return 0

        lax.fori_loop(0, nchunks, chunk_body, 0)

    lanem = lax.bitwise_and(iota16, 3)

    def p1_group(slot, s):
        idv = idbuf[slot, pl.ds(s, 16)]
        base4 = idv * 128 + lanem
        plsc.addupdate_scatter(accbuf, [4096 + idv * 4 + lanem], ones16)
        for c in range(_C):
            xv = xbuf[slot, c, pl.ds(s, 16)]
            plsc.addupdate_scatter(accbufs[c & 3], [base4 + 4 * c], xv)

    run_pass(p1_group)

    iota4 = iota16 * 4

    def merge_accbufs(j, _):
        acc = zeros16
        for m in range(4):
            idx = iota4 + (j * 64 + m)
            acc = acc + ((plsc.load_gather(accbuf, [idx])
                          + plsc.load_gather(accbuf2, [idx]))
                         + (plsc.load_gather(accbuf3, [idx])
                            + plsc.load_gather(accbuf4, [idx])))
        stage[pl.ds(j * 16, 16)] = acc
        return 0

    lax.fori_loop(0, _ACC // 16, merge_accbufs, 0)
    pltpu.sync_copy(stage, shared_all.at[sid])
    plsc.subcore_barrier()

    @pl.when(sid == 0)
    def _():
        pltpu.sync_copy(shared_all, statall)

        def red_block(j, _):
            s = j * 16
            acc = zeros16
            for r in range(16):
                acc = acc + statall[r, pl.ds(s, 16)]
            redbuf[pl.ds(s, 16)] = acc
            return 0

        lax.fori_loop(0, _ACC // 16, red_block, 0)

        for j in range(64):
            cidx = jnp.full((16,), 1024 + (j >> 1), jnp.int32)
            cntv = plsc.load_gather(redbuf, [cidx])
            inv = 1.0 / jnp.maximum(cntv, 1.0)
            meanflat[pl.ds(16 * j, 16)] = redbuf[pl.ds(16 * j, 16)] * inv
        pltpu.sync_copy(meanflat, shared_m)

    plsc.subcore_barrier()
    pltpu.sync_copy(shared_m, meanflat)

    def rep_body(j, _):
        idx = lax.shift_right_logical(iota16 + 16 * j, 2)
        meanflat4[pl.ds(16 * j, 16)] = plsc.load_gather(meanflat, [idx])
        return 0

    lax.fori_loop(0, 256, rep_body, 0)

    def p2_group(slot, s):
        idv = idbuf[slot, pl.ds(s, 16)]
        base4 = idv * 128 + lanem
        parts = [zeros16, zeros16, zeros16, zeros16]
        for c in range(_C):
            xv = xbuf[slot, c, pl.ds(s, 16)]
            mg = plsc.load_gather(meanflat4, [base4 + 4 * c])
            d = xv - mg
            parts[c & 3] = parts[c & 3] + d * d
        acc = (parts[0] + parts[1]) + (parts[2] + parts[3])
        dpx = _vsqrt(acc + 1e-12)
        hraw = jnp.maximum(dpx - _DELTA_VAR, 0.0)
        plsc.addupdate_scatter(accbuf, [4224 + idv * 4 + lanem], hraw * hraw)

    run_pass(p2_group)

    lax.fori_loop(0, _ACC // 16, merge_accbufs, 0)
    pltpu.sync_copy(stage, shared_all.at[sid])
    plsc.subcore_barrier()

    @pl.when(sid == 0)
    def _():
        pltpu.sync_copy(shared_all, statall)
        hs0 = zeros16
        hs1 = zeros16
        for r in range(16):
            hs0 = hs0 + statall[r, pl.ds(1056, 16)]
            hs1 = hs1 + statall[r, pl.ds(1072, 16)]
        safe0 = jnp.maximum(redbuf[pl.ds(1024, 16)], 1.0)
        safe1 = jnp.maximum(redbuf[pl.ds(1040, 16)], 1.0)
        var_term = (jnp.sum(hs0 / safe0) + jnp.sum(hs1 / safe1)) * (1.0 / _K)

        for c in range(_C):
            for half in range(2):
                idx = (iota16 + 16 * half) * 32 + c
                mtflat[pl.ds(c * 32 + 16 * half, 16)] = (
                    plsc.load_gather(meanflat, [idx]))

        racc0 = zeros16
        racc1 = zeros16
        for c in range(_C):
            v0 = mtflat[pl.ds(c * 32, 16)]
            v1 = mtflat[pl.ds(c * 32 + 16, 16)]
            racc0 = racc0 + v0 * v0
            racc1 = racc1 + v1 * v1
        reg_term = (jnp.sum(_vsqrt(racc0 + 1e-12))
                    + jnp.sum(_vsqrt(racc1 + 1e-12))) * (1.0 / _K)

        def pair_body(k, dist_acc):
            kb = k * 32
            a0 = zeros16
            a1 = zeros16
            for c in range(_C):
                mkc = plsc.load_gather(meanflat, [iota16 * 0 + (kb + c)])
                d0 = mtflat[pl.ds(c * 32, 16)] - mkc
                d1 = mtflat[pl.ds(c * 32 + 16, 16)] - mkc
                a0 = a0 + d0 * d0
                a1 = a1 + d1 * d1
            pd0 = _vsqrt(a0 + 1e-12)
            pd1 = _vsqrt(a1 + 1e-12)
            r0 = jnp.maximum(2.0 * _DELTA_DIST - pd0, 0.0)
            r0 = r0 * r0
            r1 = jnp.maximum(2.0 * _DELTA_DIST - pd1, 0.0)
            r1 = r1 * r1
            r0 = jnp.where(iota16 == k, 0.0, r0)
            r1 = jnp.where(iota16 + 16 == k, 0.0, r1)
            return dist_acc + jnp.sum(r0) + jnp.sum(r1)

        dist_sum = lax.fori_loop(0, _K, pair_body, jnp.float32(0.0))
        dist_term = dist_sum * (1.0 / (_K * (_K - 1)))

        loss_b = _ALPHA * var_term + _BETA * dist_term + _GAMMA * reg_term
        obuf[...] = jnp.where(iota16 == 0, loss_b, 0.0)
        pltpu.sync_copy(obuf, out_hbm.at[b])


def kernel(input_, target):
    bsz, c, h, w = input_.shape
    n = h * w
    x = input_.reshape(bsz, c, n)
    ids = target.reshape(bsz, n).astype(jnp.int32)
    nchunks = n // (16 * _CHUNK)

    mesh = plsc.VectorSubcoreMesh(core_axis_name="c", subcore_axis_name="s",
                                  num_cores=2, num_subcores=16)

    body = functools.partial(_sc_loss_body, nchunks)

    out = pl.kernel(
        body,
        out_type=jax.ShapeDtypeStruct((bsz, 16), jnp.float32),
        mesh=mesh,
        compiler_params=pltpu.CompilerParams(needs_layout_passes=False),
        scratch_types=[
            pltpu.VMEM((2, _C, _CHUNK), jnp.float32),
            pltpu.VMEM((2, _CHUNK), jnp.int32),
            pltpu.VMEM((_ACC4,), jnp.float32),
            pltpu.VMEM((_ACC4,), jnp.float32),
            pltpu.VMEM((_ACC4,), jnp.float32),
            pltpu.VMEM((_ACC4,), jnp.float32),
            pltpu.VMEM((_ACC,), jnp.float32),
            pltpu.VMEM((1024,), jnp.float32),
            pltpu.VMEM((4096,), jnp.float32),
            pltpu.VMEM((1024,), jnp.float32),
            pltpu.VMEM((_ACC,), jnp.float32),
            pltpu.VMEM((16, _ACC), jnp.float32),
            pltpu.VMEM((16,), jnp.float32),
            pltpu.SemaphoreType.DMA((2,)),
            pltpu.SemaphoreType.DMA((2,)),
            pltpu.VMEM_SHARED((16, _ACC), jnp.float32),
            pltpu.VMEM_SHARED((1024,), jnp.float32),
        ],
    )(x, ids)
    return (out[0, 0] + out[1, 0]) * 0.5


# speedup vs baseline: 1.2348x; 1.2348x over previous
import functools

import jax
import jax.numpy as jnp
from jax import lax
from jax.experimental import pallas as pl
from jax.experimental.pallas import tpu as pltpu
from jax.experimental.pallas import tpu_sc as plsc

_DELTA_VAR = 0.5
_DELTA_DIST = 1.5
_ALPHA = 1.0
_BETA = 1.0
_GAMMA = 0.001
_K = 32
_C = 32
_CHUNK = 1024
_GROUPS = _CHUNK // 16
_ACC = 1088
_ACC4 = _ACC * 4


def _vsqrt(x):
    i = plsc.bitcast(x, jnp.int32)
    i = jnp.int32(0x5F3759DF) - lax.shift_right_logical(i, 1)
    y = plsc.bitcast(i, jnp.float32)
    for _ in range(2):
        y = y * (1.5 - 0.5 * x * y * y)
    return x * y


def _sc_loss_body(nchunks, x_hbm, ids_hbm, out_hbm,
                  xbuf, idbuf, accbuf, accbuf2, accbuf3, accbuf4, stage,
                  meanflat, meanflat4, mtflat, redbuf, statall,
                  obuf, semx, semi, shared_all, shared_m):
    cid = lax.axis_index("c")
    sid = lax.axis_index("s")
    b = cid
    tile_px = nchunks * _CHUNK
    base = sid * tile_px

    zeros16 = jnp.zeros((16,), jnp.float32)
    ones16 = jnp.ones((16,), jnp.float32)
    iota16 = lax.iota(jnp.int32, 16)

    accbufs = (accbuf, accbuf2, accbuf3, accbuf4)

    def zero_body(j, _):
        sl = pl.ds(j * 16, 16)
        accbuf[sl] = zeros16
        accbuf2[sl] = zeros16
        accbuf3[sl] = zeros16
        accbuf4[sl] = zeros16
        return 0

    lax.fori_loop(0, _ACC4 // 16, zero_body, 0)

    def start_fetch(ch, slot):
        off = base + ch * _CHUNK
        pltpu.async_copy(x_hbm.at[b, :, pl.ds(off, _CHUNK)], xbuf.at[slot],
                         semx.at[slot])
        pltpu.async_copy(ids_hbm.at[b, pl.ds(off, _CHUNK)], idbuf.at[slot],
                         semi.at[slot])

    def wait_fetch(ch, slot):
        off = base + ch * _CHUNK
        pltpu.make_async_copy(x_hbm.at[b, :, pl.ds(off, _CHUNK)],
                              xbuf.at[slot], semx.at[slot]).wait()
        pltpu.make_async_copy(ids_hbm.at[b, pl.ds(off, _CHUNK)],
                              idbuf.at[slot], semi.at[slot]).wait()

    def run_pass(process_group):
        start_fetch(0, 0)

        def chunk_body(ch, _):
            slot = lax.bitwise_and(ch, 1)
            nslot = 1 - slot

            @pl.when(ch + 1 < nchunks)
            def _():
                start_fetch(ch + 1, nslot)

            wait_fetch(ch, slot)

            @plsc.parallel_loop(0, _GROUPS, unroll=8)
            def _(g):
                process_group(slot, g * 16)

            return 0

        lax.fori_loop(0, nchunks, chunk_body, 0)

    lanem = lax.bitwise_and(iota16, 3)

    def p1_group(slot, s):
        idv = idbuf[slot, pl.ds(s, 16)]
        base4 = idv * 128 + lanem
        plsc.addupdate_scatter(accbuf, [4096 + idv * 4 + lanem], ones16)
        for c in range(_C):
            xv = xbuf[slot, c, pl.ds(s, 16)]
            plsc.addupdate_scatter(accbufs[c & 3], [base4 + 4 * c], xv)

    run_pass(p1_group)

    iota4 = iota16 * 4

    def merge_accbufs(j, _):
        acc = zeros16
        for m in range(4):
            idx = iota4 + (j * 64 + m)
            acc = acc + ((plsc.load_gather(accbuf, [idx])
                          + plsc.load_gather(accbuf2, [idx]))
                         + (plsc.load_gather(accbuf3, [idx])
                            + plsc.load_gather(accbuf4, [idx])))
        stage[pl.ds(j * 16, 16)] = acc
        return 0

    lax.fori_loop(0, _ACC // 16, merge_accbufs, 0)
    pltpu.sync_copy(stage, shared_all.at[sid])
    plsc.subcore_barrier()

    @pl.when(sid == 0)
    def _():
        pltpu.sync_copy(shared_all, statall)

        def red_block(j, _):
            s = j * 16
            acc = zeros16
            for r in range(16):
                acc = acc + statall[r, pl.ds(s, 16)]
            redbuf[pl.ds(s, 16)] = acc
            return 0

        lax.fori_loop(0, _ACC // 16, red_block, 0)

        for j in range(64):
            cidx = jnp.full((16,), 1024 + (j >> 1), jnp.int32)
            cntv = plsc.load_gather(redbuf, [cidx])
            inv = 1.0 / jnp.maximum(cntv, 1.0)
            meanflat[pl.ds(16 * j, 16)] = redbuf[pl.ds(16 * j, 16)] * inv
        pltpu.sync_copy(meanflat, shared_m)

    plsc.subcore_barrier()
    pltpu.sync_copy(shared_m, meanflat)

    def rep_body(j, _):
        idx = lax.shift_right_logical(iota16 + 16 * j, 2)
        meanflat4[pl.ds(16 * j, 16)] = plsc.load_gather(meanflat, [idx])
        return 0

    lax.fori_loop(0, 256, rep_body, 0)

    def p2_group(slot, s):
        idv = idbuf[slot, pl.ds(s, 16)]
        base4 = idv * 128 + lanem
        parts = [zeros16, zeros16, zeros16, zeros16]
        for c in range(_C):
            xv = xbuf[slot, c, pl.ds(s, 16)]
            mg = plsc.load_gather(meanflat4, [base4 + 4 * c])
            d = xv - mg
            parts[c & 3] = parts[c & 3] + d * d
        acc = (parts[0] + parts[1]) + (parts[2] + parts[3])
        dpx = _vsqrt(acc + 1e-12)
        hraw = jnp.maximum(dpx - _DELTA_VAR, 0.0)
        plsc.addupdate_scatter(accbuf, [4224 + idv * 4 + lanem], hraw * hraw)

    run_pass(p2_group)

    lax.fori_loop(0, _ACC // 16, merge_accbufs, 0)
    pltpu.sync_copy(stage, shared_all.at[sid])
    plsc.subcore_barrier()

    @pl.when(sid == 0)
    def _():
        pltpu.sync_copy(shared_all, statall)
        hs0 = zeros16
        hs1 = zeros16
        for r in range(16):
            hs0 = hs0 + statall[r, pl.ds(1056, 16)]
            hs1 = hs1 + statall[r, pl.ds(1072, 16)]
        safe0 = jnp.maximum(redbuf[pl.ds(1024, 16)], 1.0)
        safe1 = jnp.maximum(redbuf[pl.ds(1040, 16)], 1.0)
        var_term = (jnp.sum(hs0 / safe0) + jnp.sum(hs1 / safe1)) * (1.0 / _K)

        for c in range(_C):
            for half in range(2):
                idx = (iota16 + 16 * half) * 32 + c
                mtflat[pl.ds(c * 32 + 16 * half, 16)] = (
                    plsc.load_gather(meanflat, [idx]))

        racc0 = zeros16
        racc1 = zeros16
        for c in range(_C):
            v0 = mtflat[pl.ds(c * 32, 16)]
            v1 = mtflat[pl.ds(c * 32 + 16, 16)]
            racc0 = racc0 + v0 * v0
            racc1 = racc1 + v1 * v1
        reg_term = (jnp.sum(_vsqrt(racc0 + 1e-12))
                    + jnp.sum(_vsqrt(racc1 + 1e-12))) * (1.0 / _K)

        def pair_body(k, dist_acc):
            kb = k * 32
            a0 = zeros16
            a1 = zeros16
            for c in range(_C):
                mkc = plsc.load_gather(meanflat, [iota16 * 0 + (kb + c)])
                d0 = mtflat[pl.ds(c * 32, 16)] - mkc
                d1 = mtflat[pl.ds(c * 32 + 16, 16)] - mkc
                a0 = a0 + d0 * d0
                a1 = a1 + d1 * d1
            pd0 = _vsqrt(a0 + 1e-12)
            pd1 = _vsqrt(a1 + 1e-12)
            r0 = jnp.maximum(2.0 * _DELTA_DIST - pd0, 0.0)
            r0 = r0 * r0
            r1 = jnp.maximum(2.0 * _DELTA_DIST - pd1, 0.0)
            r1 = r1 * r1
            r0 = jnp.where(iota16 == k, 0.0, r0)
            r1 = jnp.where(iota16 + 16 == k, 0.0, r1)
            return dist_acc + jnp.sum(r0) + jnp.sum(r1)

        dist_sum = lax.fori_loop(0, _K, pair_body, jnp.float32(0.0))
        dist_term = dist_sum * (1.0 / (_K * (_K - 1)))

        loss_b = _ALPHA * var_term + _BETA * dist_term + _GAMMA * reg_term
        obuf[...] = jnp.where(iota16 == 0, loss_b, 0.0)
        pltpu.sync_copy(obuf, out_hbm.at[b])


def kernel(input_, target):
    bsz, c, h, w = input_.shape
    n = h * w
    x = input_.reshape(bsz, c, n)
    ids = target.reshape(bsz, n).astype(jnp.int32)
    nchunks = n // (16 * _CHUNK)

    mesh = plsc.VectorSubcoreMesh(core_axis_name="c", subcore_axis_name="s",
                                  num_cores=2, num_subcores=16)

    body = functools.partial(_sc_loss_body, nchunks)

    out = pl.kernel(
        body,
        out_type=jax.ShapeDtypeStruct((bsz, 16), jnp.float32),
        mesh=mesh,
        compiler_params=pltpu.CompilerParams(needs_layout_passes=False),
        scratch_types=[
            pltpu.VMEM((2, _C, _CHUNK), jnp.float32),
            pltpu.VMEM((2, _CHUNK), jnp.int32),
            pltpu.VMEM((_ACC4,), jnp.float32),
            pltpu.VMEM((_ACC4,), jnp.float32),
            pltpu.VMEM((_ACC4,), jnp.float32),
            pltpu.VMEM((_ACC4,), jnp.float32),
            pltpu.VMEM((_ACC,), jnp.float32),
            pltpu.VMEM((1024,), jnp.float32),
            pltpu.VMEM((4096,), jnp.float32),
            pltpu.VMEM((1024,), jnp.float32),
            pltpu.VMEM((_ACC,), jnp.float32),
            pltpu.VMEM((16, _ACC), jnp.float32),
            pltpu.VMEM((16,), jnp.float32),
            pltpu.SemaphoreType.DMA((2,)),
            pltpu.SemaphoreType.DMA((2,)),
            pltpu.VMEM_SHARED((16, _ACC), jnp.float32),
            pltpu.VMEM_SHARED((1024,), jnp.float32),
        ],
    )(x, ids)
    return (out[0, 0] + out[1, 0]) * 0.5
